# baseline (speedup 1.0000x reference)
import dataclasses
import functools

import jax
import jax.numpy as jnp
from jax import lax
from jax.experimental import pallas as pl
from jax.experimental.pallas import tpu as pltpu
from jax.experimental.pallas import tpu_sc as plsc

_CUTOFF_SQ = 1.5 ** 2
_K = 128
_BATCH = 128
_N = 16384
_NCH = _N // 16
_HBINS = 1024
_NW = 32
_RPW = _BATCH // _NW

_mesh = plsc.VectorSubcoreMesh(core_axis_name="c", subcore_axis_name="s")
_cp = pltpu.CompilerParams()
if "needs_layout_passes" in pltpu.CompilerParams.__dataclass_fields__:
    _cp = dataclasses.replace(_cp, needs_layout_passes=False)


@functools.partial(
    pl.kernel,
    mesh=_mesh,
    compiler_params=_cp,
    out_type=jax.ShapeDtypeStruct((_BATCH, 3 * _K), jnp.float32),
    scratch_types=[
        pltpu.VMEM((3 * _N,), jnp.float32),
        pltpu.VMEM((_N,), jnp.float32),
        pltpu.VMEM((_HBINS,), jnp.int32),
        pltpu.VMEM((_N + 16,), jnp.float32),
        pltpu.VMEM((_N + 16,), jnp.int32),
        pltpu.VMEM((_K + 16,), jnp.float32),
        pltpu.VMEM((_K + 16,), jnp.int32),
        pltpu.VMEM((3 * _BATCH + 16,), jnp.float32),
        pltpu.VMEM((3 * _K,), jnp.float32),
        pltpu.VMEM((_NCH,), jnp.int32),
        pltpu.VMEM((_NCH + 16,), jnp.int32),
        pltpu.VMEM((_NCH + 16,), jnp.int32),
        pltpu.SemaphoreType.DMA,
        pltpu.SemaphoreType.DMA,
        pltpu.SemaphoreType.DMA,
        pltpu.SemaphoreType.DMA,
    ],
)
def _sc_select(coords_hbm, ref_hbm, out_hbm,
               crow, drow, hist, cd, ci, sd, si, refv, outv,
               cnt, nzid, nzbase, sem0, sem1, sem2, sem3):
    wid = lax.axis_index("s") * 2 + lax.axis_index("c")
    lane = lax.iota(jnp.int32, 16)
    ones_i = jnp.ones((16,), jnp.int32)
    zeros_i = jnp.zeros((16,), jnp.int32)

    pltpu.sync_copy(ref_hbm, refv)

    sems = (sem0, sem1, sem2, sem3)
    nq = 4
    q_sz = 3 * _N // nq
    q_ch = _NCH // nq

    @pl.loop(0, _RPW)
    def _row(r):
        b = wid * _RPW + r
        row0 = b * (3 * _N)
        copies = [
            pltpu.async_copy(coords_hbm.at[pl.ds(row0 + q * q_sz, q_sz)],
                             crow.at[pl.ds(q * q_sz, q_sz)], sems[q])
            for q in range(nq)
        ]

        rvec = refv[pl.ds(b * 3, 16)]
        rx = jnp.full((16,), rvec[0], jnp.float32)
        ry = jnp.full((16,), rvec[1], jnp.float32)
        rz = jnp.full((16,), rvec[2], jnp.float32)

        @pl.loop(0, _HBINS // 16)
        def _hz(h):
            hist[pl.ds(h * 16, 16)] = zeros_i

        for q in range(nq):
            copies[q].wait()

            @plsc.parallel_loop(q * q_ch, (q + 1) * q_ch, unroll=4)
            def _p1(c):
                fp = c * 48 + lane * 3
                x = plsc.load_gather(crow, [fp])
                y = plsc.load_gather(crow, [fp + 1])
                z = plsc.load_gather(crow, [fp + 2])
                dx = x - rx
                dy = y - ry
                dz = z - rz
                d = (dx * dx + dy * dy) + dz * dz
                drow[pl.ds(c * 16, 16)] = d
                bins = lax.shift_right_logical(plsc.bitcast(d, jnp.int32), 21)
                plsc.addupdate_scatter(hist, [bins], ones_i)

        def _scan(i, carry):
            total, bsel_v, cless_v = carry
            h = hist[pl.ds(i * 16, 16)]
            cum = plsc.cumsum(h) + total
            mlt = cum < _K
            bsel_v = bsel_v + plsc.all_reduce_population_count(mlt)
            cless_v = jnp.maximum(cless_v, jnp.where(mlt, cum, 0))
            return cum[15], bsel_v, cless_v

        _, bsel_v, _cless_v = lax.fori_loop(
            0, _HBINS // 16, _scan, (jnp.int32(0), zeros_i, zeros_i))
        bin_sel = bsel_v[0]
        bin_sel_v = jnp.full((16,), bin_sel, jnp.int32)

        @plsc.parallel_loop(0, _NCH, unroll=4)
        def _p2a(c):
            d = drow[pl.ds(c * 16, 16)]
            bins = lax.shift_right_logical(plsc.bitcast(d, jnp.int32), 21)
            keep = bins <= bin_sel_v
            plsc.store_compressed(cd.at[pl.ds(c * 16, 16)], d, mask=keep)
            plsc.store_compressed(ci.at[pl.ds(c * 16, 16)], c * 16 + lane,
                                  mask=keep)
            pc = plsc.all_reduce_population_count(keep)
            cv = jnp.full((16,), c, jnp.int32)
            plsc.store_scatter(cnt, [cv], pc, mask=lane == 0)

        def _p2b(g, carry):
            total, off_v = carry
            cv = cnt[pl.ds(g * 16, 16)]
            cs = plsc.cumsum(cv) + total
            base_v = cs - cv
            nz = cv > 0
            off_s = off_v[0]
            plsc.store_compressed(nzid.at[pl.ds(off_s, 16)], g * 16 + lane,
                                  mask=nz)
            plsc.store_compressed(nzbase.at[pl.ds(off_s, 16)], base_v, mask=nz)
            off_v = off_v + plsc.all_reduce_population_count(nz)
            return cs[15], off_v

        s_cnt, m_v = lax.fori_loop(0, _NCH // 16, _p2b,
                                   (jnp.int32(0), zeros_i))
        m_cnt = m_v[0]

        @pl.loop(0, m_cnt)
        def _p2c(j):
            cid = nzid[pl.ds(j, 16)][0]
            bse = nzbase[pl.ds(j, 16)][0]
            dvv = cd[pl.ds(cid * 16, 16)]
            ivv = ci[pl.ds(cid * 16, 16)]
            cd[pl.ds(bse, 16)] = dvv
            ci[pl.ds(bse, 16)] = ivv

        cd[pl.ds(s_cnt, 16)] = jnp.full((16,), jnp.inf, jnp.float32)
        ci[pl.ds(s_cnt, 16)] = jnp.full((16,), jnp.int32(1 << 30), jnp.int32)
        nvec = (s_cnt + 15) // 16

        @plsc.parallel_loop(0, s_cnt)
        def _rank(i):
            iv16 = jnp.full((16,), i, jnp.int32)
            div = plsc.load_gather(cd, [iv16])
            iiv = plsc.load_gather(ci, [iv16])

            def _inner(j, acc):
                dd = cd[pl.ds(j * 16, 16)]
                xi = ci[pl.ds(j * 16, 16)]
                less = (dd < div) | ((dd == div) & (xi < iiv))
                return acc + jnp.where(less, 1, 0)

            acc = lax.fori_loop(0, nvec, _inner, zeros_i)
            rank_vec = lax.rev(plsc.cumsum(acc), (0,))
            keepm = (lane == 0) & (rank_vec < _K)
            plsc.store_scatter(sd, [rank_vec], div, mask=keepm)
            plsc.store_scatter(si, [rank_vec], iiv, mask=keepm)

        b3 = jnp.full((16,), b * 3, jnp.int32)

        @pl.loop(0, 3 * _K // 16)
        def _out(v):
            fpos = v * 16 + lane
            slot = fpos // 3
            comp = fpos - slot * 3
            p = plsc.load_gather(si, [slot])
            dsel = plsc.load_gather(sd, [slot])
            val = plsc.load_gather(crow, [p * 3 + comp])
            rc = plsc.load_gather(refv, [b3 + comp])
            res = jnp.where(dsel <= _CUTOFF_SQ, val - rc,
                            jnp.zeros((16,), jnp.float32))
            outv[pl.ds(v * 16, 16)] = res

        pltpu.sync_copy(outv, out_hbm.at[b])


def kernel(coords, ref):
    batch, n, _ = coords.shape
    coords_flat = coords.reshape(batch * 3 * n)
    ref_flat = jnp.pad(ref.reshape(-1), (0, 16))
    out = _sc_select(coords_flat, ref_flat)
    return out.reshape(batch, _K, 3)


# speedup vs baseline: 34.6828x; 34.6828x over previous
import dataclasses
import functools

import jax
import jax.numpy as jnp
from jax import lax
from jax.experimental import pallas as pl
from jax.experimental.pallas import tpu as pltpu
from jax.experimental.pallas import tpu_sc as plsc

_CUTOFF_SQ = 1.5 ** 2
_K = 128
_BATCH = 128
_N = 16384
_NCH = _N // 16
_HBINS = 1024
_NW = 32
_RPW = _BATCH // _NW

_mesh = plsc.VectorSubcoreMesh(core_axis_name="c", subcore_axis_name="s")
_cp = pltpu.CompilerParams()
if "needs_layout_passes" in pltpu.CompilerParams.__dataclass_fields__:
    _cp = dataclasses.replace(_cp, needs_layout_passes=False)


@functools.partial(
    pl.kernel,
    mesh=_mesh,
    compiler_params=_cp,
    out_type=jax.ShapeDtypeStruct((_BATCH, 3 * _K), jnp.float32),
    scratch_types=[
        pltpu.VMEM((3 * _N // 128, 128), jnp.float32),
        pltpu.VMEM((_N,), jnp.float32),
        pltpu.VMEM((_HBINS,), jnp.int32),
        pltpu.VMEM((_N + 16,), jnp.float32),
        pltpu.VMEM((_N + 16,), jnp.int32),
        pltpu.VMEM((_K + 16,), jnp.float32),
        pltpu.VMEM((_K + 16,), jnp.int32),
        pltpu.VMEM((3 * _BATCH + 16,), jnp.float32),
        pltpu.VMEM((3 * _K,), jnp.float32),
        pltpu.VMEM((_NCH,), jnp.int32),
        pltpu.VMEM((_NCH + 16,), jnp.int32),
        pltpu.VMEM((_NCH + 16,), jnp.int32),
        pltpu.SemaphoreType.DMA,
        pltpu.SemaphoreType.DMA,
        pltpu.SemaphoreType.DMA,
        pltpu.SemaphoreType.DMA,
    ],
)
def _sc_select(coords_hbm, ref_hbm, out_hbm,
               crow, drow, hist, cd, ci, sd, si, refv, outv,
               cnt, nzid, nzbase, sem0, sem1, sem2, sem3):
    wid = lax.axis_index("s") * 2 + lax.axis_index("c")
    lane = lax.iota(jnp.int32, 16)
    ones_i = jnp.ones((16,), jnp.int32)
    zeros_i = jnp.zeros((16,), jnp.int32)

    pltpu.sync_copy(ref_hbm, refv)

    sems = (sem0, sem1, sem2, sem3)
    nq = 4
    q_rows = 3 * _N // 128 // nq
    q_ch = _NCH // nq

    @pl.loop(0, _RPW)
    def _row(r):
        b = wid * _RPW + r
        copies = [
            pltpu.async_copy(
                coords_hbm.at[b].at[pl.ds(q * q_rows, q_rows)],
                crow.at[pl.ds(q * q_rows, q_rows)], sems[q])
            for q in range(nq)
        ]

        rvec = refv[pl.ds(b * 3, 16)]
        rx = jnp.full((16,), rvec[0], jnp.float32)
        ry = jnp.full((16,), rvec[1], jnp.float32)
        rz = jnp.full((16,), rvec[2], jnp.float32)

        @pl.loop(0, _HBINS // 16)
        def _hz(h):
            hist[pl.ds(h * 16, 16)] = zeros_i

        for q in range(nq):
            copies[q].wait()

            @plsc.parallel_loop(q * q_ch, (q + 1) * q_ch, unroll=4)
            def _p1(c):
                fp = c * 48 + lane * 3
                x = plsc.load_gather(crow, [fp >> 7, fp & 127])
                fp1 = fp + 1
                y = plsc.load_gather(crow, [fp1 >> 7, fp1 & 127])
                fp2 = fp + 2
                z = plsc.load_gather(crow, [fp2 >> 7, fp2 & 127])
                dx = x - rx
                dy = y - ry
                dz = z - rz
                d = (dx * dx + dy * dy) + dz * dz
                drow[pl.ds(c * 16, 16)] = d
                bins = lax.shift_right_logical(plsc.bitcast(d, jnp.int32), 21)
                plsc.addupdate_scatter(hist, [bins], ones_i)

        def _scan(i, carry):
            total, bsel_v, cless_v = carry
            h = hist[pl.ds(i * 16, 16)]
            cum = plsc.cumsum(h) + total
            mlt = cum < _K
            bsel_v = bsel_v + plsc.all_reduce_population_count(mlt)
            cless_v = jnp.maximum(cless_v, jnp.where(mlt, cum, 0))
            return cum[15], bsel_v, cless_v

        _, bsel_v, _cless_v = lax.fori_loop(
            0, _HBINS // 16, _scan, (jnp.int32(0), zeros_i, zeros_i))
        bin_sel = bsel_v[0]
        bin_sel_v = jnp.full((16,), bin_sel, jnp.int32)

        @plsc.parallel_loop(0, _NCH, unroll=4)
        def _p2a(c):
            d = drow[pl.ds(c * 16, 16)]
            bins = lax.shift_right_logical(plsc.bitcast(d, jnp.int32), 21)
            keep = bins <= bin_sel_v
            plsc.store_compressed(cd.at[pl.ds(c * 16, 16)], d, mask=keep)
            plsc.store_compressed(ci.at[pl.ds(c * 16, 16)], c * 16 + lane,
                                  mask=keep)
            pc = plsc.all_reduce_population_count(keep)
            cv = jnp.full((16,), c, jnp.int32)
            plsc.store_scatter(cnt, [cv], pc, mask=lane == 0)

        def _p2b(g, carry):
            total, off_v = carry
            cv = cnt[pl.ds(g * 16, 16)]
            cs = plsc.cumsum(cv) + total
            base_v = cs - cv
            nz = cv > 0
            off_s = off_v[0]
            plsc.store_compressed(nzid.at[pl.ds(off_s, 16)], g * 16 + lane,
                                  mask=nz)
            plsc.store_compressed(nzbase.at[pl.ds(off_s, 16)], base_v, mask=nz)
            off_v = off_v + plsc.all_reduce_population_count(nz)
            return cs[15], off_v

        s_cnt, m_v = lax.fori_loop(0, _NCH // 16, _p2b,
                                   (jnp.int32(0), zeros_i))
        m_cnt = m_v[0]

        @pl.loop(0, m_cnt)
        def _p2c(j):
            cid = nzid[pl.ds(j, 16)][0]
            bse = nzbase[pl.ds(j, 16)][0]
            dvv = cd[pl.ds(cid * 16, 16)]
            ivv = ci[pl.ds(cid * 16, 16)]
            cd[pl.ds(bse, 16)] = dvv
            ci[pl.ds(bse, 16)] = ivv

        cd[pl.ds(s_cnt, 16)] = jnp.full((16,), jnp.inf, jnp.float32)
        ci[pl.ds(s_cnt, 16)] = jnp.full((16,), jnp.int32(1 << 30), jnp.int32)
        nvec = (s_cnt + 15) // 16

        @plsc.parallel_loop(0, s_cnt)
        def _rank(i):
            iv16 = jnp.full((16,), i, jnp.int32)
            div = plsc.load_gather(cd, [iv16])
            iiv = plsc.load_gather(ci, [iv16])

            def _inner(j, acc):
                dd = cd[pl.ds(j * 16, 16)]
                xi = ci[pl.ds(j * 16, 16)]
                less = (dd < div) | ((dd == div) & (xi < iiv))
                return acc + jnp.where(less, 1, 0)

            acc = lax.fori_loop(0, nvec, _inner, zeros_i)
            rank_vec = lax.rev(plsc.cumsum(acc), (0,))
            keepm = (lane == 0) & (rank_vec < _K)
            plsc.store_scatter(sd, [rank_vec], div, mask=keepm)
            plsc.store_scatter(si, [rank_vec], iiv, mask=keepm)

        b3 = jnp.full((16,), b * 3, jnp.int32)

        @pl.loop(0, 3 * _K // 16)
        def _out(v):
            fpos = v * 16 + lane
            slot = fpos // 3
            comp = fpos - slot * 3
            p = plsc.load_gather(si, [slot])
            dsel = plsc.load_gather(sd, [slot])
            fsel = p * 3 + comp
            val = plsc.load_gather(crow, [fsel >> 7, fsel & 127])
            rc = plsc.load_gather(refv, [b3 + comp])
            res = jnp.where(dsel <= _CUTOFF_SQ, val - rc,
                            jnp.zeros((16,), jnp.float32))
            outv[pl.ds(v * 16, 16)] = res

        pltpu.sync_copy(outv, out_hbm.at[b])


def kernel(coords, ref):
    batch, n, _ = coords.shape
    coords_flat = coords.reshape(batch, 3 * n // 128, 128)
    ref_flat = jnp.pad(ref.reshape(-1), (0, 16))
    out = _sc_select(coords_flat, ref_flat)
    return out.reshape(batch, _K, 3)
